# baseline (speedup 1.0000x reference)
import functools

import jax
import jax.numpy as jnp
from jax import lax
from jax.experimental import pallas as pl
from jax.experimental.pallas import tpu as pltpu
from jax.experimental.pallas import tpu_sc as plsc

VOCAB = 1000000
D = 32
B = 4096
S = 50
L = 16
NC, NS = 2, 16
NW = NC * NS
ROWS_W = B // NW
ROWS_P = 64
NPASS = ROWS_W // ROWS_P
TOK_P = ROWS_P * S
CHUNK = 128
NCH = TOK_P // CHUNK
KW = (S + L - 1) // L


def _body(desc_flat, word_hbm, weight_hbm, out_hbm,
          idx_v, emb_v, w_v, wexp_v, out_v, gsem, wsem):
    wid = lax.axis_index("s") * NC + lax.axis_index("c")
    iota = lax.iota(jnp.int32, L)

    for p in range(NPASS):
        row0 = wid * ROWS_W + p * ROWS_P
        irow0 = wid * (ROWS_W * S // CHUNK) + p * NCH

        pltpu.sync_copy(desc_flat.at[pl.ds(irow0 * CHUNK, NCH * CHUNK)], idx_v)

        copies = []
        for i in range(NCH):
            copies.append(pltpu.async_copy(
                word_hbm.at[idx_v.at[pl.ds(i * CHUNK, CHUNK)]],
                emb_v.at[pl.ds(i * CHUNK, CHUNK), :], gsem))
            copies.append(pltpu.async_copy(
                weight_hbm.at[idx_v.at[pl.ds(i * CHUNK, CHUNK)]],
                w_v.at[pl.ds(i * CHUNK, CHUNK)], wsem))
        for c in copies:
            c.wait()

        def row_body(r, _):
            base = r * S
            wvecs = []
            for k in range(KW):
                idxs = jnp.minimum(base + k * L + iota, base + S - 1)
                wvecs.append(plsc.load_gather(w_v, [idxs]))
            masks = [(k * L + iota) < S for k in range(KW)]
            mvec = jnp.where(masks[0], wvecs[0], -1e30)
            for k in range(1, KW):
                mvec = jnp.maximum(mvec, jnp.where(masks[k], wvecs[k], -1e30))
            mx = jnp.max(mvec)
            svec = jnp.zeros((L,), jnp.float32)
            wbase = r * (KW * L)
            for k in range(KW):
                e_k = jnp.where(masks[k], jnp.exp(wvecs[k] - mx), 0.0)
                wexp_v[pl.ds(wbase + k * L, L)] = e_k
                svec = svec + e_k
            inv = jnp.ones((L,), jnp.float32) / lax.broadcast(jnp.sum(svec), (L,))
            acc0 = jnp.zeros((L,), jnp.float32)
            acc1 = jnp.zeros((L,), jnp.float32)
            for j in range(S):
                wb = plsc.load_gather(wexp_v, [lax.broadcast(wbase + j, (L,))])
                acc0 = acc0 + wb * emb_v[base + j, pl.ds(0, L)]
                acc1 = acc1 + wb * emb_v[base + j, pl.ds(L, L)]
            out_v[r, pl.ds(0, L)] = acc0 * inv
            out_v[r, pl.ds(L, L)] = acc1 * inv
            return _

        lax.fori_loop(0, ROWS_P, row_body, 0)

        pltpu.sync_copy(out_v, out_hbm.at[pl.ds(row0, ROWS_P), :])


@jax.jit
def _run(desc_flat, word_table, weight_table):
    mesh = plsc.VectorSubcoreMesh(core_axis_name="c", subcore_axis_name="s")
    return pl.kernel(
        _body,
        out_type=jax.ShapeDtypeStruct((B, D), jnp.float32),
        mesh=mesh,
        scratch_types=[
            pltpu.VMEM((NCH * CHUNK,), jnp.int32),
            pltpu.VMEM((TOK_P, D), jnp.float32),
            pltpu.VMEM((TOK_P,), jnp.float32),
            pltpu.VMEM((ROWS_P * KW * L,), jnp.float32),
            pltpu.VMEM((ROWS_P, D), jnp.float32),
            pltpu.SemaphoreType.DMA,
            pltpu.SemaphoreType.DMA,
        ],
        compiler_params=pltpu.CompilerParams(
            needs_layout_passes=False, use_tc_tiling_on_sc=False),
    )(desc_flat, word_table, weight_table)


def kernel(desc, word_table, weight_table):
    return _run(desc.reshape(B * S), word_table, weight_table.reshape(VOCAB))
